# Initial kernel scaffold; baseline (speedup 1.0000x reference)
import jax
import jax.numpy as jnp
from jax.experimental import pallas as pl


def kernel(x, edge_index, batch, W, b):
    raise NotImplementedError("write your pallas kernel here")


# speedup vs baseline: 2.9438x; 2.9438x over previous
import functools

import jax
import jax.numpy as jnp
from jax import lax
from jax.experimental import pallas as pl
from jax.experimental.pallas import tpu as pltpu
from jax.experimental.pallas import tpu_sc as plsc

N = 10000
NP = 10240
E = 320000
EP = 327680
D = 128
K = 10
G = 64

NC = 2
NS = 16
CH = 128
ET = EP // (NC * NS)
NCHUNK = ET // CH
RPT = NP // NS

_MESH = plsc.VectorSubcoreMesh(core_axis_name="c", subcore_axis_name="s")


@functools.partial(
    pl.kernel,
    out_type=jax.ShapeDtypeStruct((NC, NP), jnp.float32),
    mesh=_MESH,
    scratch_types=[
        pltpu.VMEM((CH,), jnp.int32),
        pltpu.VMEM((CH,), jnp.float32),
        pltpu.VMEM_SHARED((NP,), jnp.float32),
    ],
)
def _sc_degree(row_hbm, ones_hbm, zeros1_hbm, deg_hbm, rowv, onesv, acc):
    c = lax.axis_index("c")
    s = lax.axis_index("s")
    pltpu.sync_copy(zeros1_hbm, acc.at[pl.ds(s * RPT, RPT)])
    pltpu.sync_copy(ones_hbm, onesv)
    plsc.subcore_barrier()
    base0 = c * (EP // NC) + s * ET

    def step(i, carry):
        base = base0 + i * CH
        pltpu.sync_copy(row_hbm.at[pl.ds(base, CH)], rowv)
        pltpu.sync_copy(onesv, acc.at[rowv], add=True)
        return carry

    lax.fori_loop(0, NCHUNK, step, 0)
    plsc.subcore_barrier()
    pltpu.sync_copy(acc.at[pl.ds(s * RPT, RPT)], deg_hbm.at[c, pl.ds(s * RPT, RPT)])


@functools.partial(
    pl.kernel,
    out_type=jax.ShapeDtypeStruct((NC, NP, D), jnp.float32),
    mesh=_MESH,
    scratch_types=[
        pltpu.VMEM((CH,), jnp.int32),
        pltpu.VMEM((CH,), jnp.int32),
        pltpu.VMEM((CH, D), jnp.float32),
        pltpu.VMEM_SHARED((NP, D), jnp.float32),
        pltpu.SemaphoreType.DMA,
    ],
)
def _sc_spmm(y_hbm, col_hbm, row_hbm, zeros_hbm, s_hbm,
             colv, rowv, rowsv, acc, sem):
    c = lax.axis_index("c")
    s = lax.axis_index("s")
    pltpu.sync_copy(zeros_hbm, acc.at[pl.ds(s * RPT, RPT)])
    plsc.subcore_barrier()
    base0 = c * (EP // NC) + s * ET

    def step(i, carry):
        base = base0 + i * CH
        pltpu.sync_copy(col_hbm.at[pl.ds(base, CH)], colv)
        pltpu.async_copy(y_hbm.at[colv], rowsv, sem).wait()
        pltpu.sync_copy(row_hbm.at[pl.ds(base, CH)], rowv)
        pltpu.sync_copy(rowsv, acc.at[rowv], add=True)
        return carry

    lax.fori_loop(0, NCHUNK, step, 0)
    plsc.subcore_barrier()
    pltpu.sync_copy(acc.at[pl.ds(s * RPT, RPT)],
                    s_hbm.at[c, pl.ds(s * RPT, RPT)])


_BLK = 2048


def _tc_prep_body(dg0_ref, dg1_ref, x_ref, dv_ref, y0_ref):
    deg = dg0_ref[...] + dg1_ref[...]
    dinv = jnp.where(deg > 0.0,
                     1.0 / jnp.sqrt(jnp.maximum(deg, 1e-12)), 0.0)
    dv = jnp.broadcast_to(dinv, x_ref.shape)
    dv_ref[...] = dv
    y0_ref[...] = dv * x_ref[...]


def _tc_prep(dg0, dg1, x):
    nb = NP // _BLK
    return pl.pallas_call(
        _tc_prep_body,
        grid=(nb,),
        in_specs=[
            pl.BlockSpec((_BLK, 1), lambda i: (i, 0)),
            pl.BlockSpec((_BLK, 1), lambda i: (i, 0)),
            pl.BlockSpec((_BLK, D), lambda i: (i, 0)),
        ],
        out_specs=[
            pl.BlockSpec((_BLK, D), lambda i: (i, 0)),
            pl.BlockSpec((_BLK, D), lambda i: (i, 0)),
        ],
        out_shape=[
            jax.ShapeDtypeStruct((NP, D), jnp.float32),
            jax.ShapeDtypeStruct((NP, D), jnp.float32),
        ],
    )(dg0, dg1, x)


def _tc_combine_first_body(s_ref, dv_ref, tx_ref, y_ref):
    dv = dv_ref[...]
    tx = -1.0 * dv * (s_ref[0] + s_ref[1])
    tx_ref[...] = tx
    y_ref[...] = dv * tx


def _tc_combine_body(s_ref, dv_ref, txp_ref, tx_ref, y_ref):
    dv = dv_ref[...]
    tx = -2.0 * dv * (s_ref[0] + s_ref[1]) - txp_ref[...]
    tx_ref[...] = tx
    y_ref[...] = dv * tx


def _tc_combine(s, dv, txp=None):
    nb = NP // _BLK
    in_specs = [
        pl.BlockSpec((NC, _BLK, D), lambda i: (0, i, 0)),
        pl.BlockSpec((_BLK, D), lambda i: (i, 0)),
    ]
    args = [s, dv]
    body = _tc_combine_first_body
    if txp is not None:
        in_specs.append(pl.BlockSpec((_BLK, D), lambda i: (i, 0)))
        args.append(txp)
        body = _tc_combine_body
    return pl.pallas_call(
        body,
        grid=(nb,),
        in_specs=in_specs,
        out_specs=[
            pl.BlockSpec((_BLK, D), lambda i: (i, 0)),
            pl.BlockSpec((_BLK, D), lambda i: (i, 0)),
        ],
        out_shape=[
            jax.ShapeDtypeStruct((NP, D), jnp.float32),
            jax.ShapeDtypeStruct((NP, D), jnp.float32),
        ],
    )(*args)


_FBLK = 512
_FNB = NP // _FBLK


def _tc_final_body(*refs):
    tx_refs = refs[:K]
    w_ref, b_ref, batch_ref = refs[K], refs[K + 1], refs[K + 2]
    out_ref = refs[K + 3]
    acc_ref, cnt_ref = refs[K + 4], refs[K + 5]
    i = pl.program_id(0)

    h = jnp.zeros((_FBLK, D), jnp.float32)
    for k in range(K):
        h = h + jnp.dot(tx_refs[k][...], w_ref[k],
                        preferred_element_type=jnp.float32)
    h = h + b_ref[...]

    bb = batch_ref[...]
    gid = lax.broadcasted_iota(jnp.int32, (G, _FBLK), 0)
    oht = (jnp.broadcast_to(bb, (G, _FBLK)) == gid).astype(jnp.float32)
    contrib = jnp.dot(oht, h, preferred_element_type=jnp.float32)
    csum = jnp.sum(oht, axis=1, keepdims=True)
    cnt_c = jnp.broadcast_to(csum, (G, D))

    @pl.when(i == 0)
    def _():
        acc_ref[...] = jnp.zeros((G, D), jnp.float32)
        cnt_ref[...] = jnp.zeros((G, D), jnp.float32)

    acc_ref[...] += contrib
    cnt_ref[...] += cnt_c

    @pl.when(i == _FNB - 1)
    def _():
        out_ref[...] = acc_ref[...] / jnp.maximum(cnt_ref[...], 1.0)


def _tc_final(txs, W, b2, batch2):
    in_specs = [pl.BlockSpec((_FBLK, D), lambda i: (i, 0)) for _ in range(K)]
    in_specs += [
        pl.BlockSpec((K, D, D), lambda i: (0, 0, 0)),
        pl.BlockSpec((1, D), lambda i: (0, 0)),
        pl.BlockSpec((1, _FBLK), lambda i: (0, i)),
    ]
    return pl.pallas_call(
        _tc_final_body,
        grid=(_FNB,),
        in_specs=in_specs,
        out_specs=pl.BlockSpec((G, D), lambda i: (0, 0)),
        out_shape=jax.ShapeDtypeStruct((G, D), jnp.float32),
        scratch_shapes=[
            pltpu.VMEM((G, D), jnp.float32),
            pltpu.VMEM((G, D), jnp.float32),
        ],
    )(*txs, W, b2, batch2)


def kernel(x, edge_index, batch, W, b):
    xp = jnp.zeros((NP, D), jnp.float32).at[:N].set(x)
    npad = EP - E
    row = jnp.concatenate(
        [edge_index[0], (N + (jnp.arange(npad, dtype=jnp.int32) % 16))])
    col = jnp.concatenate([edge_index[1], jnp.zeros((npad,), jnp.int32)])
    batch2 = jnp.full((1, NP), G, jnp.int32).at[0, :N].set(batch)

    ones_ch = jnp.ones((CH,), jnp.float32)
    zeros1 = jnp.zeros((RPT,), jnp.float32)
    zeros2 = jnp.zeros((RPT, D), jnp.float32)
    b2 = b.reshape(1, D)

    deg = _sc_degree(row, ones_ch, zeros1)
    dg0 = deg[0].reshape(NP, 1)
    dg1 = deg[1].reshape(NP, 1)
    dv, y = _tc_prep(dg0, dg1, xp)

    txs = [xp]
    txp = None
    for _ in range(1, K):
        s = _sc_spmm(y, col, row, zeros2)
        tx, y = _tc_combine(s, dv, txp)
        txp = txs[-1]
        txs.append(tx)

    return _tc_final(txs, W, b2, batch2)


# speedup vs baseline: 4.3109x; 1.4644x over previous
import functools

import jax
import jax.numpy as jnp
from jax import lax
from jax.experimental import pallas as pl
from jax.experimental.pallas import tpu as pltpu
from jax.experimental.pallas import tpu_sc as plsc

N = 10000
NP = 10240
E = 320000
EP = 327680
D = 128
K = 10
G = 64

NC = 2
NS = 16
CH = 128
ET = EP // (NC * NS)
NCHUNK = ET // CH
RPT = NP // NS

_MESH = plsc.VectorSubcoreMesh(core_axis_name="c", subcore_axis_name="s")

NB = 4
NCROW = EP // CH
CPT = NCHUNK


@functools.partial(
    pl.kernel,
    out_type=jax.ShapeDtypeStruct((NC, NP), jnp.float32),
    mesh=_MESH,
    scratch_types=[
        pltpu.VMEM((CPT, CH), jnp.int32),
        pltpu.VMEM((CH,), jnp.float32),
        pltpu.VMEM_SHARED((NP,), jnp.float32),
        pltpu.SemaphoreType.DMA,
    ],
)
def _sc_degree(row_hbm, ones_hbm, zeros1_hbm, deg_hbm, rowv, onesv, acc, sem):
    c = lax.axis_index("c")
    s = lax.axis_index("s")
    pltpu.sync_copy(zeros1_hbm, acc.at[pl.ds(s * RPT, RPT)])
    pltpu.sync_copy(ones_hbm, onesv)
    cbase = c * (NCROW // NC) + s * CPT
    pltpu.sync_copy(row_hbm.at[pl.ds(cbase, CPT)], rowv)
    plsc.subcore_barrier()

    def rnd(j, carry):
        for b in range(NB):
            pltpu.async_copy(onesv, acc.at[rowv.at[j * NB + b]], sem, add=True)
        for b in range(NB):
            pltpu.make_async_copy(ones_hbm, onesv, sem).wait()
        return carry

    lax.fori_loop(0, CPT // NB, rnd, 0)
    plsc.subcore_barrier()
    pltpu.sync_copy(acc.at[pl.ds(s * RPT, RPT)], deg_hbm.at[c, pl.ds(s * RPT, RPT)])


DH = D // 2
ROUNDS = CPT // NB


@functools.partial(
    pl.kernel,
    out_type=jax.ShapeDtypeStruct((2, NC, NP, DH), jnp.float32),
    mesh=_MESH,
    scratch_types=[
        pltpu.VMEM((CPT, CH), jnp.int32),
        pltpu.VMEM((CPT, CH), jnp.int32),
        pltpu.VMEM((2, NB, CH, DH), jnp.float32),
        pltpu.VMEM_SHARED((NP, DH), jnp.float32),
        pltpu.SemaphoreType.DMA((2, NB)),
        pltpu.SemaphoreType.DMA((2, NB)),
    ],
    compiler_params=pltpu.CompilerParams(use_tc_tiling_on_sc=False),
)
def _sc_spmm(ya_hbm, yb_hbm, col_hbm, row_hbm, zeros_hbm, s_hbm,
             colv, rowv, bufs, acc, gsem, ssem):
    c = lax.axis_index("c")
    s = lax.axis_index("s")
    cbase = c * (NCROW // NC) + s * CPT
    pltpu.sync_copy(col_hbm.at[pl.ds(cbase, CPT)], colv)
    pltpu.sync_copy(row_hbm.at[pl.ds(cbase, CPT)], rowv)

    for h, y_hbm in ((0, ya_hbm), (1, yb_hbm)):
        pltpu.sync_copy(zeros_hbm, acc.at[pl.ds(s * RPT, RPT)])
        plsc.subcore_barrier()
        for b in range(NB):
            pltpu.async_copy(y_hbm.at[colv.at[b]], bufs.at[0, b],
                             gsem.at[0, b])

        def round_body(j, p):
            pn = 1 - p
            for b in range(NB):
                @pl.when(j >= 1)
                def _():
                    pltpu.make_async_copy(
                        bufs.at[pn, b], acc.at[rowv.at[0]],
                        ssem.at[pn, b]).wait()
            for b in range(NB):
                @pl.when(j + 1 < ROUNDS)
                def _():
                    pltpu.async_copy(
                        y_hbm.at[colv.at[(j + 1) * NB + b]],
                        bufs.at[pn, b], gsem.at[pn, b])
            for b in range(NB):
                i = j * NB + b
                pltpu.make_async_copy(
                    y_hbm.at[colv.at[i]], bufs.at[p, b], gsem.at[p, b]).wait()
                pltpu.async_copy(bufs.at[p, b], acc.at[rowv.at[i]],
                                 ssem.at[p, b], add=True)

        def dbl(j2, carry):
            round_body(2 * j2, 0)
            round_body(2 * j2 + 1, 1)
            return carry

        lax.fori_loop(0, ROUNDS // 2, dbl, 0)
        plast = (ROUNDS - 1) % 2
        for b in range(NB):
            pltpu.make_async_copy(
                bufs.at[plast, b], acc.at[rowv.at[0]],
                ssem.at[plast, b]).wait()
        plsc.subcore_barrier()
        pltpu.sync_copy(acc.at[pl.ds(s * RPT, RPT)],
                        s_hbm.at[h, c, pl.ds(s * RPT, RPT)])
        plsc.subcore_barrier()


_BLK = 2048


def _tc_prep_body(dg0_ref, dg1_ref, x_ref, dv_ref, ya_ref, yb_ref):
    deg = dg0_ref[...] + dg1_ref[...]
    dinv = jnp.where(deg > 0.0,
                     1.0 / jnp.sqrt(jnp.maximum(deg, 1e-12)), 0.0)
    dv = jnp.broadcast_to(dinv, x_ref.shape)
    dv_ref[...] = dv
    y = dv * x_ref[...]
    ya_ref[...] = y[:, :DH]
    yb_ref[...] = y[:, DH:]


def _tc_prep(dg0, dg1, x):
    nb = NP // _BLK
    return pl.pallas_call(
        _tc_prep_body,
        grid=(nb,),
        in_specs=[
            pl.BlockSpec((_BLK, 1), lambda i: (i, 0)),
            pl.BlockSpec((_BLK, 1), lambda i: (i, 0)),
            pl.BlockSpec((_BLK, D), lambda i: (i, 0)),
        ],
        out_specs=[
            pl.BlockSpec((_BLK, D), lambda i: (i, 0)),
            pl.BlockSpec((_BLK, DH), lambda i: (i, 0)),
            pl.BlockSpec((_BLK, DH), lambda i: (i, 0)),
        ],
        out_shape=[
            jax.ShapeDtypeStruct((NP, D), jnp.float32),
            jax.ShapeDtypeStruct((NP, DH), jnp.float32),
            jax.ShapeDtypeStruct((NP, DH), jnp.float32),
        ],
    )(dg0, dg1, x)


def _tc_combine_first_body(s_ref, dv_ref, tx_ref, ya_ref, yb_ref):
    dv = dv_ref[...]
    ssum = jnp.concatenate(
        [s_ref[0, 0] + s_ref[0, 1], s_ref[1, 0] + s_ref[1, 1]], axis=1)
    tx = -1.0 * dv * ssum
    tx_ref[...] = tx
    y = dv * tx
    ya_ref[...] = y[:, :DH]
    yb_ref[...] = y[:, DH:]


def _tc_combine_body(s_ref, dv_ref, txp_ref, tx_ref, ya_ref, yb_ref):
    dv = dv_ref[...]
    ssum = jnp.concatenate(
        [s_ref[0, 0] + s_ref[0, 1], s_ref[1, 0] + s_ref[1, 1]], axis=1)
    tx = -2.0 * dv * ssum - txp_ref[...]
    tx_ref[...] = tx
    y = dv * tx
    ya_ref[...] = y[:, :DH]
    yb_ref[...] = y[:, DH:]


def _tc_combine(s, dv, txp=None):
    nb = NP // _BLK
    in_specs = [
        pl.BlockSpec((2, NC, _BLK, DH), lambda i: (0, 0, i, 0)),
        pl.BlockSpec((_BLK, D), lambda i: (i, 0)),
    ]
    args = [s, dv]
    body = _tc_combine_first_body
    if txp is not None:
        in_specs.append(pl.BlockSpec((_BLK, D), lambda i: (i, 0)))
        args.append(txp)
        body = _tc_combine_body
    return pl.pallas_call(
        body,
        grid=(nb,),
        in_specs=in_specs,
        out_specs=[
            pl.BlockSpec((_BLK, D), lambda i: (i, 0)),
            pl.BlockSpec((_BLK, DH), lambda i: (i, 0)),
            pl.BlockSpec((_BLK, DH), lambda i: (i, 0)),
        ],
        out_shape=[
            jax.ShapeDtypeStruct((NP, D), jnp.float32),
            jax.ShapeDtypeStruct((NP, DH), jnp.float32),
            jax.ShapeDtypeStruct((NP, DH), jnp.float32),
        ],
    )(*args)


_FBLK = 512
_FNB = NP // _FBLK


def _tc_final_body(*refs):
    tx_refs = refs[:K]
    w_ref, b_ref, batch_ref = refs[K], refs[K + 1], refs[K + 2]
    out_ref = refs[K + 3]
    acc_ref, cnt_ref = refs[K + 4], refs[K + 5]
    i = pl.program_id(0)

    h = jnp.zeros((_FBLK, D), jnp.float32)
    for k in range(K):
        h = h + jnp.dot(tx_refs[k][...], w_ref[k],
                        preferred_element_type=jnp.float32)
    h = h + b_ref[...]

    bb = batch_ref[...]
    gid = lax.broadcasted_iota(jnp.int32, (G, _FBLK), 0)
    oht = (jnp.broadcast_to(bb, (G, _FBLK)) == gid).astype(jnp.float32)
    contrib = jnp.dot(oht, h, preferred_element_type=jnp.float32)
    csum = jnp.sum(oht, axis=1, keepdims=True)
    cnt_c = jnp.broadcast_to(csum, (G, D))

    @pl.when(i == 0)
    def _():
        acc_ref[...] = jnp.zeros((G, D), jnp.float32)
        cnt_ref[...] = jnp.zeros((G, D), jnp.float32)

    acc_ref[...] += contrib
    cnt_ref[...] += cnt_c

    @pl.when(i == _FNB - 1)
    def _():
        out_ref[...] = acc_ref[...] / jnp.maximum(cnt_ref[...], 1.0)


def _tc_final(txs, W, b2, batch2):
    in_specs = [pl.BlockSpec((_FBLK, D), lambda i: (i, 0)) for _ in range(K)]
    in_specs += [
        pl.BlockSpec((K, D, D), lambda i: (0, 0, 0)),
        pl.BlockSpec((1, D), lambda i: (0, 0)),
        pl.BlockSpec((1, _FBLK), lambda i: (0, i)),
    ]
    return pl.pallas_call(
        _tc_final_body,
        grid=(_FNB,),
        in_specs=in_specs,
        out_specs=pl.BlockSpec((G, D), lambda i: (0, 0)),
        out_shape=jax.ShapeDtypeStruct((G, D), jnp.float32),
        scratch_shapes=[
            pltpu.VMEM((G, D), jnp.float32),
            pltpu.VMEM((G, D), jnp.float32),
        ],
    )(*txs, W, b2, batch2)


def kernel(x, edge_index, batch, W, b):
    xp = jnp.zeros((NP, D), jnp.float32).at[:N].set(x)
    npad = EP - E
    row = jnp.concatenate(
        [edge_index[0], (N + (jnp.arange(npad, dtype=jnp.int32) % 16))]
    ).reshape(NCROW, CH)
    col = jnp.concatenate(
        [edge_index[1], jnp.zeros((npad,), jnp.int32)]).reshape(NCROW, CH)
    batch2 = jnp.full((1, NP), G, jnp.int32).at[0, :N].set(batch)

    ones_ch = jnp.ones((CH,), jnp.float32)
    zeros1 = jnp.zeros((RPT,), jnp.float32)
    zeros2 = jnp.zeros((RPT, DH), jnp.float32)
    b2 = b.reshape(1, D)

    deg = _sc_degree(row, ones_ch, zeros1)
    dg0 = deg[0].reshape(NP, 1)
    dg1 = deg[1].reshape(NP, 1)
    dv, ya, yb = _tc_prep(dg0, dg1, xp)

    txs = [xp]
    txp = None
    for _ in range(1, K):
        s = _sc_spmm(ya, yb, col, row, zeros2)
        tx, ya, yb = _tc_combine(s, dv, txp)
        txp = txs[-1]
        txs.append(tx)

    return _tc_final(txs, W, b2, batch2)
